# initial kernel scaffold (unmeasured)
import jax
import jax.numpy as jnp
from jax import lax
from jax.experimental import pallas as pl
from jax.experimental.pallas import tpu as pltpu

N_DEV = 4
M_BLK = 1024
K_BLK = 1024
N_OUT = 8192
NT = 2048
T = N_OUT // NT


def kernel(x, w_mat):

    def body(x_hbm, w_hbm, out_ref, xg, wbuf, amax_my, amax_all,
             send_sems, recv_sems, a_send_sems, a_recv_sems, w_sems, loc_sem):
        my = lax.axis_index("i")

        loc_copy = pltpu.make_async_copy(
            x_hbm.at[pl.ds(my * M_BLK, M_BLK), :], xg.at[0], loc_sem)
        loc_copy.start()

        barrier_sem = pltpu.get_barrier_semaphore()
        for d in range(1, N_DEV):
            pl.semaphore_signal(
                barrier_sem, inc=1,
                device_id=((my + d) % N_DEV,),
                device_id_type=pl.DeviceIdType.MESH,
            )
        pl.semaphore_wait(barrier_sem, N_DEV - 1)

        sends = []
        for d in (1, 3, 2):
            dest = (my + d) % N_DEV
            rdma = pltpu.make_async_remote_copy(
                src_ref=x_hbm.at[pl.ds(dest * M_BLK, M_BLK), :],
                dst_ref=xg.at[N_DEV - d],
                send_sem=send_sems.at[d],
                recv_sem=recv_sems.at[N_DEV - d],
                device_id=(dest,),
                device_id_type=pl.DeviceIdType.MESH,
            )
            rdma.start()
            sends.append(rdma)

        os_order = (0, 1, 3, 2)
        flat = [(o, t) for o in os_order for t in range(T)]

        def w_dma(f):
            o, t = flat[f]
            j = (my + o) % N_DEV
            return pltpu.make_async_copy(
                w_hbm.at[pl.ds(j * K_BLK, K_BLK), pl.ds(t * NT, NT)],
                wbuf.at[f % 2], w_sems.at[f % 2])

        w_dma(0).start()
        for f, (o, t) in enumerate(flat):
            if f + 1 < len(flat):
                w_dma(f + 1).start()
            if t == 0:
                if o == 0:
                    loc_copy.wait()
                else:
                    pl.semaphore_wait(recv_sems.at[o], 1)
            w_dma(f).wait()
            acc = jnp.dot(xg[o], wbuf[f % 2],
                          preferred_element_type=jnp.float32)
            if o == 0:
                out_ref[:, pl.ds(t * NT, NT)] = acc
            else:
                out_ref[:, pl.ds(t * NT, NT)] += acc

        local_amax = jnp.max(jnp.abs(out_ref[:, :]))
        amax_my[:, :] = jnp.full((8, 128), local_amax, jnp.float32)
        amax_all[0] = amax_my[:, :]
        a_sends = []
        for d in (1, 3, 2):
            rdma = pltpu.make_async_remote_copy(
                src_ref=amax_my,
                dst_ref=amax_all.at[N_DEV - d],
                send_sem=a_send_sems.at[d],
                recv_sem=a_recv_sems.at[N_DEV - d],
                device_id=((my + d) % N_DEV,),
                device_id_type=pl.DeviceIdType.MESH,
            )
            rdma.start()
            a_sends.append(rdma)
        for o in (1, 2, 3):
            pl.semaphore_wait(a_recv_sems.at[o], 1)
        g_amax = jnp.max(amax_all[:, :, :])

        scale = g_amax / 127.0
        inv = 1.0 / scale
        for t in range(T):
            y = out_ref[:, pl.ds(t * NT, NT)]
            q = jnp.clip(jnp.round(y * inv), -127.0, 127.0)
            out_ref[:, pl.ds(t * NT, NT)] = q * scale

        for rdma in sends:
            rdma.wait_send()
        for rdma in a_sends:
            rdma.wait_send()

    return pl.pallas_call(
        body,
        out_shape=jax.ShapeDtypeStruct((M_BLK, N_OUT), jnp.float32),
        in_specs=[
            pl.BlockSpec(memory_space=pltpu.ANY),
            pl.BlockSpec(memory_space=pltpu.ANY),
        ],
        out_specs=pl.BlockSpec(memory_space=pltpu.VMEM),
        scratch_shapes=[
            pltpu.VMEM((N_DEV, M_BLK, K_BLK), jnp.float32),
            pltpu.VMEM((2, K_BLK, NT), jnp.float32),
            pltpu.VMEM((8, 128), jnp.float32),
            pltpu.VMEM((N_DEV, 8, 128), jnp.float32),
            pltpu.SemaphoreType.DMA((N_DEV,)),
            pltpu.SemaphoreType.DMA((N_DEV,)),
            pltpu.SemaphoreType.DMA((N_DEV,)),
            pltpu.SemaphoreType.DMA((N_DEV,)),
            pltpu.SemaphoreType.DMA((2,)),
            pltpu.SemaphoreType.DMA,
        ],
        compiler_params=pltpu.CompilerParams(
            collective_id=0,
            vmem_limit_bytes=128 * 1024 * 1024,
        ),
    )(x, w_mat)


# baseline (device time: 163484 ns/iter reference)
import jax
import jax.numpy as jnp
from jax import lax
from jax.experimental import pallas as pl
from jax.experimental.pallas import tpu as pltpu

N_DEV = 4
M_BLK = 1024
K_BLK = 1024
N_OUT = 8192
NT = 1024
T = N_OUT // NT


def kernel(x, w_mat):

    def body(x_hbm, w_hbm, out_ref, xg, wbuf, amax_my, amax_all,
             send_sems, recv_sems, a_send_sems, a_recv_sems, w_sems, loc_sem):
        my = lax.axis_index("i")

        loc_copy = pltpu.make_async_copy(
            x_hbm.at[pl.ds(my * M_BLK, M_BLK), :], xg.at[0], loc_sem)
        loc_copy.start()

        barrier_sem = pltpu.get_barrier_semaphore()
        for d in range(1, N_DEV):
            pl.semaphore_signal(
                barrier_sem, inc=1,
                device_id=((my + d) % N_DEV,),
                device_id_type=pl.DeviceIdType.MESH,
            )
        pl.semaphore_wait(barrier_sem, N_DEV - 1)

        sends = []
        for d in (1, 3, 2):
            dest = (my + d) % N_DEV
            rdma = pltpu.make_async_remote_copy(
                src_ref=x_hbm.at[pl.ds(dest * M_BLK, M_BLK), :],
                dst_ref=xg.at[N_DEV - d],
                send_sem=send_sems.at[d],
                recv_sem=recv_sems.at[N_DEV - d],
                device_id=(dest,),
                device_id_type=pl.DeviceIdType.MESH,
            )
            rdma.start()
            sends.append(rdma)

        os_order = (0, 1, 3, 2)
        flat = [(o, t) for o in os_order for t in range(T)]

        def w_dma(f):
            o, t = flat[f]
            j = (my + o) % N_DEV
            return pltpu.make_async_copy(
                w_hbm.at[pl.ds(j * K_BLK, K_BLK), pl.ds(t * NT, NT)],
                wbuf.at[f % 2], w_sems.at[f % 2])

        w_dma(0).start()
        for f, (o, t) in enumerate(flat):
            if f + 1 < len(flat):
                w_dma(f + 1).start()
            if t == 0:
                if o == 0:
                    loc_copy.wait()
                else:
                    recv = pltpu.make_async_remote_copy(
                        src_ref=x_hbm.at[pl.ds(0, M_BLK), :],
                        dst_ref=xg.at[o],
                        send_sem=send_sems.at[o],
                        recv_sem=recv_sems.at[o],
                        device_id=(my,),
                        device_id_type=pl.DeviceIdType.MESH,
                    )
                    recv.wait_recv()
            w_dma(f).wait()
            acc = jnp.dot(xg[o], wbuf[f % 2],
                          preferred_element_type=jnp.float32)
            if o == 0:
                out_ref[:, pl.ds(t * NT, NT)] = acc
            else:
                out_ref[:, pl.ds(t * NT, NT)] += acc

        local_amax = jnp.float32(0.0)
        for t in range(T):
            local_amax = jnp.maximum(
                local_amax, jnp.max(jnp.abs(out_ref[:, pl.ds(t * NT, NT)])))
        amax_my[:, :] = jnp.full((8, 128), local_amax, jnp.float32)
        amax_all[0] = amax_my[:, :]
        a_sends = []
        for d in (1, 3, 2):
            rdma = pltpu.make_async_remote_copy(
                src_ref=amax_my,
                dst_ref=amax_all.at[N_DEV - d],
                send_sem=a_send_sems.at[d],
                recv_sem=a_recv_sems.at[N_DEV - d],
                device_id=((my + d) % N_DEV,),
                device_id_type=pl.DeviceIdType.MESH,
            )
            rdma.start()
            a_sends.append(rdma)
        for o in (1, 2, 3):
            recv = pltpu.make_async_remote_copy(
                src_ref=amax_my,
                dst_ref=amax_all.at[o],
                send_sem=a_send_sems.at[o],
                recv_sem=a_recv_sems.at[o],
                device_id=(my,),
                device_id_type=pl.DeviceIdType.MESH,
            )
            recv.wait_recv()
        g_amax = jnp.max(amax_all[:, :, :])

        scale = g_amax / 127.0
        inv = 1.0 / scale
        for t in range(T):
            y = out_ref[:, pl.ds(t * NT, NT)]
            q = jnp.clip(jnp.round(y * inv), -127.0, 127.0)
            out_ref[:, pl.ds(t * NT, NT)] = q * scale

        for rdma in sends:
            rdma.wait_send()
        for rdma in a_sends:
            rdma.wait_send()

    return pl.pallas_call(
        body,
        out_shape=jax.ShapeDtypeStruct((M_BLK, N_OUT), jnp.float32),
        in_specs=[
            pl.BlockSpec(memory_space=pl.ANY),
            pl.BlockSpec(memory_space=pl.ANY),
        ],
        out_specs=pl.BlockSpec(memory_space=pltpu.VMEM),
        scratch_shapes=[
            pltpu.VMEM((N_DEV, M_BLK, K_BLK), jnp.float32),
            pltpu.VMEM((2, K_BLK, NT), jnp.float32),
            pltpu.VMEM((8, 128), jnp.float32),
            pltpu.VMEM((N_DEV, 8, 128), jnp.float32),
            pltpu.SemaphoreType.DMA((N_DEV,)),
            pltpu.SemaphoreType.DMA((N_DEV,)),
            pltpu.SemaphoreType.DMA((N_DEV,)),
            pltpu.SemaphoreType.DMA((N_DEV,)),
            pltpu.SemaphoreType.DMA((2,)),
            pltpu.SemaphoreType.DMA,
        ],
        compiler_params=pltpu.CompilerParams(
            collective_id=0,
            vmem_limit_bytes=128 * 1024 * 1024,
        ),
    )(x, w_mat)


# device time: 146851 ns/iter; 1.1133x vs baseline; 1.1133x over previous
import contextlib

import jax
import jax.numpy as jnp
from jax import lax
from jax.experimental import pallas as pl
from jax.experimental.pallas import tpu as pltpu

PROFILE_SCOPES = False


def _scope(name):
    return jax.named_scope(name) if PROFILE_SCOPES else contextlib.nullcontext()


N_DEV = 4
M_BLK = 1024
K_BLK = 1024
N_OUT = 8192
NT = 1024
T = N_OUT // NT


def kernel(x, w_mat):

    def body(x_hbm, w_hbm, out_ref, xf, xs, xg, xb, wbuf, amax_my, amax_all,
             send_sems, recv_sems, a_send_sems, a_recv_sems, w_sems, loc_sem):
        my = lax.axis_index("i")

        barrier_sem = pltpu.get_barrier_semaphore()
        for d in range(1, N_DEV):
            pl.semaphore_signal(
                barrier_sem, inc=1,
                device_id=((my + d) % N_DEV,),
                device_id_type=pl.DeviceIdType.MESH,
            )
        pl.semaphore_wait(barrier_sem, N_DEV - 1)

        sends = []
        for d in (1, 3, 2, 0):
            r = (my + d) % N_DEV
            cp = pltpu.make_async_copy(
                x_hbm.at[pl.ds(r * M_BLK, M_BLK), :], xf, loc_sem)
            cp.start()
            cp.wait()
            if d == 0:
                break
            xs[d - 1] = xf[:, :].astype(jnp.bfloat16)
            rdma = pltpu.make_async_remote_copy(
                src_ref=xs.at[d - 1],
                dst_ref=xg.at[3 - d],
                send_sem=send_sems.at[d - 1],
                recv_sem=recv_sems.at[3 - d],
                device_id=(r,),
                device_id_type=pl.DeviceIdType.MESH,
            )
            rdma.start()
            sends.append(rdma)

        os_order = (0, 3, 1, 2)
        flat = [(o, t) for o in os_order for t in range(T)]

        def w_dma(f):
            o, t = flat[f]
            j = (my + o) % N_DEV
            return pltpu.make_async_copy(
                w_hbm.at[pl.ds(j * K_BLK, K_BLK), pl.ds(t * NT, NT)],
                wbuf.at[f % 2], w_sems.at[f % 2])

        w_dma(0).start()
        for f, (o, t) in enumerate(flat):
            if f + 1 < len(flat):
                w_dma(f + 1).start()
            if t == 0 and o != 0:
                with _scope(f"wait_recv#o={o}"):
                    recv = pltpu.make_async_remote_copy(
                        src_ref=xs.at[0],
                        dst_ref=xg.at[o - 1],
                        send_sem=send_sems.at[o - 1],
                        recv_sem=recv_sems.at[o - 1],
                        device_id=(my,),
                        device_id_type=pl.DeviceIdType.MESH,
                    )
                    recv.wait_recv()
                xb[:, :] = xg[o - 1].astype(jnp.float32)
            with _scope(f"mm#o={o}_t={t}"):
                w_dma(f).wait()
                lhs = xf[:, :] if o == 0 else xb[:, :]
                acc = jnp.dot(lhs, wbuf[f % 2],
                              preferred_element_type=jnp.float32)
                if o == 0:
                    out_ref[:, pl.ds(t * NT, NT)] = acc
                else:
                    out_ref[:, pl.ds(t * NT, NT)] += acc

        with _scope("local_amax"):
            local_amax = jnp.float32(0.0)
            for t in range(T):
                local_amax = jnp.maximum(
                    local_amax, jnp.max(jnp.abs(out_ref[:, pl.ds(t * NT, NT)])))

        with _scope("amax_exchange"):
            amax_my[:, :] = jnp.full((8, 128), local_amax, jnp.float32)
            amax_all[0] = amax_my[:, :]
            a_sends = []
            for d in (1, 3, 2):
                rdma = pltpu.make_async_remote_copy(
                    src_ref=amax_my,
                    dst_ref=amax_all.at[N_DEV - d],
                    send_sem=a_send_sems.at[d],
                    recv_sem=a_recv_sems.at[N_DEV - d],
                    device_id=((my + d) % N_DEV,),
                    device_id_type=pl.DeviceIdType.MESH,
                )
                rdma.start()
                a_sends.append(rdma)
            for o in (1, 2, 3):
                recv = pltpu.make_async_remote_copy(
                    src_ref=amax_my,
                    dst_ref=amax_all.at[o],
                    send_sem=a_send_sems.at[o],
                    recv_sem=a_recv_sems.at[o],
                    device_id=(my,),
                    device_id_type=pl.DeviceIdType.MESH,
                )
                recv.wait_recv()
            g_amax = jnp.max(amax_all[:, :, :])

        with _scope("qdq"):
            scale = g_amax / 127.0
            inv = 1.0 / scale
            for t in range(T):
                y = out_ref[:, pl.ds(t * NT, NT)]
                q = jnp.clip(jnp.round(y * inv), -127.0, 127.0)
                out_ref[:, pl.ds(t * NT, NT)] = q * scale

        for rdma in sends:
            rdma.wait_send()
        for rdma in a_sends:
            rdma.wait_send()

    return pl.pallas_call(
        body,
        out_shape=jax.ShapeDtypeStruct((M_BLK, N_OUT), jnp.float32),
        in_specs=[
            pl.BlockSpec(memory_space=pl.ANY),
            pl.BlockSpec(memory_space=pl.ANY),
        ],
        out_specs=pl.BlockSpec(memory_space=pltpu.VMEM),
        scratch_shapes=[
            pltpu.VMEM((M_BLK, K_BLK), jnp.float32),
            pltpu.VMEM((N_DEV - 1, M_BLK, K_BLK), jnp.bfloat16),
            pltpu.VMEM((N_DEV - 1, M_BLK, K_BLK), jnp.bfloat16),
            pltpu.VMEM((M_BLK, K_BLK), jnp.float32),
            pltpu.VMEM((2, K_BLK, NT), jnp.float32),
            pltpu.VMEM((8, 128), jnp.float32),
            pltpu.VMEM((N_DEV, 8, 128), jnp.float32),
            pltpu.SemaphoreType.DMA((N_DEV,)),
            pltpu.SemaphoreType.DMA((N_DEV,)),
            pltpu.SemaphoreType.DMA((N_DEV,)),
            pltpu.SemaphoreType.DMA((N_DEV,)),
            pltpu.SemaphoreType.DMA((2,)),
            pltpu.SemaphoreType.DMA,
        ],
        compiler_params=pltpu.CompilerParams(
            collective_id=0,
            vmem_limit_bytes=128 * 1024 * 1024,
        ),
    )(x, w_mat)


# device time: 134991 ns/iter; 1.2111x vs baseline; 1.0879x over previous
import contextlib

import jax
import jax.numpy as jnp
from jax import lax
from jax.experimental import pallas as pl
from jax.experimental.pallas import tpu as pltpu

PROFILE_SCOPES = False


def _scope(name):
    return jax.named_scope(name) if PROFILE_SCOPES else contextlib.nullcontext()


N_DEV = 4
M_BLK = 1024
K_BLK = 1024
N_OUT = 8192
NT = 1024
T = N_OUT // NT


def kernel(x, w_mat):

    def body(x_hbm, w_hbm, out_hbm, acc, xf, xs, xg, xb, wbuf, amax_my,
             amax_all, send_sems, recv_sems, a_send_sems, a_recv_sems,
             w_sems, out_sems, loc_sem):
        my = lax.axis_index("i")

        barrier_sem = pltpu.get_barrier_semaphore()
        for d in range(1, N_DEV):
            pl.semaphore_signal(
                barrier_sem, inc=1,
                device_id=((my + d) % N_DEV,),
                device_id_type=pl.DeviceIdType.MESH,
            )
        pl.semaphore_wait(barrier_sem, N_DEV - 1)

        sends = []
        for d in (1, 3, 2, 0):
            r = (my + d) % N_DEV
            cp = pltpu.make_async_copy(
                x_hbm.at[pl.ds(r * M_BLK, M_BLK), :], xf, loc_sem)
            cp.start()
            cp.wait()
            if d == 0:
                break
            xs[d - 1] = xf[:, :].astype(jnp.bfloat16)
            rdma = pltpu.make_async_remote_copy(
                src_ref=xs.at[d - 1],
                dst_ref=xg.at[3 - d],
                send_sem=send_sems.at[d - 1],
                recv_sem=recv_sems.at[3 - d],
                device_id=(r,),
                device_id_type=pl.DeviceIdType.MESH,
            )
            rdma.start()
            sends.append(rdma)

        os_order = (0, 3, 1, 2)
        flat = [(o, t) for o in os_order for t in range(T)]

        def w_dma(f):
            o, t = flat[f]
            j = (my + o) % N_DEV
            return pltpu.make_async_copy(
                w_hbm.at[pl.ds(j * K_BLK, K_BLK), pl.ds(t * NT, NT)],
                wbuf.at[f % 2], w_sems.at[f % 2])

        w_dma(0).start()
        local_amax = jnp.float32(0.0)
        for f, (o, t) in enumerate(flat):
            if f + 1 < len(flat):
                w_dma(f + 1).start()
            if t == 0 and o != 0:
                with _scope(f"wait_recv#o={o}"):
                    recv = pltpu.make_async_remote_copy(
                        src_ref=xs.at[0],
                        dst_ref=xg.at[o - 1],
                        send_sem=send_sems.at[o - 1],
                        recv_sem=recv_sems.at[o - 1],
                        device_id=(my,),
                        device_id_type=pl.DeviceIdType.MESH,
                    )
                    recv.wait_recv()
                xb[:, :] = xg[o - 1].astype(jnp.float32)
            with _scope(f"mm#o={o}_t={t}"):
                w_dma(f).wait()
                lhs = xf[:, :] if o == 0 else xb[:, :]
                val = jnp.dot(lhs, wbuf[f % 2],
                              preferred_element_type=jnp.float32)
                if o == 0:
                    acc[:, pl.ds(t * NT, NT)] = val
                else:
                    val = acc[:, pl.ds(t * NT, NT)] + val
                    acc[:, pl.ds(t * NT, NT)] = val
                if o == os_order[-1]:
                    local_amax = jnp.maximum(
                        local_amax, jnp.max(jnp.abs(val)))

        with _scope("amax_exchange"):
            amax_my[:, :] = jnp.full((8, 128), local_amax, jnp.float32)
            amax_all[0] = amax_my[:, :]
            a_sends = []
            for d in (1, 3, 2):
                rdma = pltpu.make_async_remote_copy(
                    src_ref=amax_my,
                    dst_ref=amax_all.at[N_DEV - d],
                    send_sem=a_send_sems.at[d],
                    recv_sem=a_recv_sems.at[N_DEV - d],
                    device_id=((my + d) % N_DEV,),
                    device_id_type=pl.DeviceIdType.MESH,
                )
                rdma.start()
                a_sends.append(rdma)
            for o in (1, 2, 3):
                recv = pltpu.make_async_remote_copy(
                    src_ref=amax_my,
                    dst_ref=amax_all.at[o],
                    send_sem=a_send_sems.at[o],
                    recv_sem=a_recv_sems.at[o],
                    device_id=(my,),
                    device_id_type=pl.DeviceIdType.MESH,
                )
                recv.wait_recv()
            g_amax = jnp.max(amax_all[:, :, :])

        with _scope("qdq"):
            scale = g_amax / 127.0
            inv = 1.0 / scale
            out_dmas = []
            for t in range(T):
                y = acc[:, pl.ds(t * NT, NT)]
                q = jnp.clip(jnp.round(y * inv), -127.0, 127.0)
                acc[:, pl.ds(t * NT, NT)] = q * scale
                if t >= 2:
                    out_dmas[t - 2].wait()
                dma = pltpu.make_async_copy(
                    acc.at[:, pl.ds(t * NT, NT)],
                    out_hbm.at[:, pl.ds(t * NT, NT)],
                    out_sems.at[t % 2])
                dma.start()
                out_dmas.append(dma)
            for dma in out_dmas[-2:]:
                dma.wait()

        for rdma in sends:
            rdma.wait_send()
        for rdma in a_sends:
            rdma.wait_send()

    return pl.pallas_call(
        body,
        out_shape=jax.ShapeDtypeStruct((M_BLK, N_OUT), jnp.float32),
        in_specs=[
            pl.BlockSpec(memory_space=pl.ANY),
            pl.BlockSpec(memory_space=pl.ANY),
        ],
        out_specs=pl.BlockSpec(memory_space=pl.ANY),
        scratch_shapes=[
            pltpu.VMEM((M_BLK, N_OUT), jnp.float32),
            pltpu.VMEM((M_BLK, K_BLK), jnp.float32),
            pltpu.VMEM((N_DEV - 1, M_BLK, K_BLK), jnp.bfloat16),
            pltpu.VMEM((N_DEV - 1, M_BLK, K_BLK), jnp.bfloat16),
            pltpu.VMEM((M_BLK, K_BLK), jnp.float32),
            pltpu.VMEM((2, K_BLK, NT), jnp.float32),
            pltpu.VMEM((8, 128), jnp.float32),
            pltpu.VMEM((N_DEV, 8, 128), jnp.float32),
            pltpu.SemaphoreType.DMA((N_DEV,)),
            pltpu.SemaphoreType.DMA((N_DEV,)),
            pltpu.SemaphoreType.DMA((N_DEV,)),
            pltpu.SemaphoreType.DMA((N_DEV,)),
            pltpu.SemaphoreType.DMA((2,)),
            pltpu.SemaphoreType.DMA((2,)),
            pltpu.SemaphoreType.DMA,
        ],
        compiler_params=pltpu.CompilerParams(
            collective_id=0,
            vmem_limit_bytes=128 * 1024 * 1024,
        ),
    )(x, w_mat)
